# baseline (speedup 1.0000x reference)
import jax
import jax.numpy as jnp
import numpy as np
from jax import lax
from jax.experimental import pallas as pl
from jax.experimental.pallas import tpu as pltpu
from jax.experimental.pallas import tpu_sc as plsc

N_NODES = 10000
N_EDGES = 320000
NF = 128
NG = 64
N_CONV = 3
CUTOFF = 5.0
NPG = 1000
XD = 16

_OFFSETS = np.linspace(0.0, CUTOFF, NG).astype(np.float32)
_COEFF = float(-0.5 / (_OFFSETS[1] - _OFFSETS[0]) ** 2)
_LN2 = float(np.log(2.0))

NC = 2
NS = 16
NW = NC * NS
CH = 64
N_CHUNKS = N_EDGES // CH
ROUNDS = -(-N_CHUNKS // NW)
WB = 40
N_WCHUNKS = N_NODES // WB
W_ROUNDS = -(-N_WCHUNKS // NS)

import functools


@functools.cache
def _sc_mesh():
    return plsc.VectorSubcoreMesh(
        core_axis_name="c", subcore_axis_name="s", num_cores=NC, num_subcores=NS
    )


def _ssp(x):
    return jax.nn.softplus(x) - _LN2


EPT = N_EDGES // NW


def _sc_edge_d2(x0, x1, x2, src, dst):
    def body(x0_h, x1_h, x2_h, src_h, dst_h, out_h, xv0, xv1, xv2, si, di, d2v):
        cid = lax.axis_index("c")
        sid = lax.axis_index("s")
        wid = sid * NC + cid
        base = wid * EPT
        pltpu.sync_copy(x0_h, xv0)
        pltpu.sync_copy(x1_h, xv1)
        pltpu.sync_copy(x2_h, xv2)
        pltpu.sync_copy(src_h.at[pl.ds(base, EPT)], si)
        pltpu.sync_copy(dst_h.at[pl.ds(base, EPT)], di)

        def step(j, carry):
            s = pl.ds(j * 16, 16)
            ia = si[s]
            ib = di[s]
            df0 = plsc.load_gather(xv0, [ia]) - plsc.load_gather(xv0, [ib])
            df1 = plsc.load_gather(xv1, [ia]) - plsc.load_gather(xv1, [ib])
            df2 = plsc.load_gather(xv2, [ia]) - plsc.load_gather(xv2, [ib])
            d2v[s] = df0 * df0 + df1 * df1 + df2 * df2
            return carry

        lax.fori_loop(0, EPT // 16, step, 0)
        pltpu.sync_copy(d2v, out_h.at[pl.ds(base, EPT)])

    f = pl.kernel(
        body,
        out_type=jax.ShapeDtypeStruct((E_PAD,), jnp.float32),
        mesh=_sc_mesh(),
        scratch_types=[
            pltpu.VMEM((N_NODES,), jnp.float32),
            pltpu.VMEM((N_NODES,), jnp.float32),
            pltpu.VMEM((N_NODES,), jnp.float32),
            pltpu.VMEM((EPT,), jnp.int32),
            pltpu.VMEM((EPT,), jnp.int32),
            pltpu.VMEM((EPT,), jnp.float32),
        ],
        compiler_params=pltpu.CompilerParams(needs_layout_passes=False),
        name="sc_edge_d2",
    )
    return f(x0, x1, x2, src, dst)


CPT = -(-N_CHUNKS // NW)
IBC = 8
NBLK = -(-CPT // IBC)
IDXPAD = NW * (CPT + IBC) * CH


def _sc_aggregate(h, w, src, dst):
    def body(h_h, w_h, src_h, dst_h, out_h,
             si_blk, di_blk,
             si0, di0, hd0, hs0, wv0, si1, di1, hd1, hs1, wv1,
             agg_sh, gsem0, gsem1, ssem0, ssem1):
        cid = lax.axis_index("c")
        sid = lax.axis_index("s")
        wid = sid * NC + cid
        c0 = wid * CPT
        cend = jnp.minimum(c0 + CPT, N_CHUNKS)
        zero16 = jnp.zeros((16,), jnp.float32)
        bufs = ((si0, di0, hd0, hs0, wv0, gsem0, ssem0),
                (si1, di1, hd1, hs1, wv1, gsem1, ssem1))

        def zbody(e, carry):
            for j in range(NF // 16):
                hd0[e, pl.ds(j * 16, 16)] = zero16
            return carry

        lax.fori_loop(0, WB, zbody, 0)
        for k in range(W_ROUNDS):
            zchunk = k * NS + sid

            @pl.when(zchunk < N_WCHUNKS)
            def _():
                pltpu.sync_copy(
                    hd0.at[pl.ds(0, WB)],
                    agg_sh.at[pl.ds(zchunk * WB, WB)],
                )

        def refill(blk):
            off = (c0 + blk * IBC) * CH
            pltpu.sync_copy(src_h.at[pl.ds(off, IBC * CH)], si_blk)
            pltpu.sync_copy(dst_h.at[pl.ds(off, IBC * CH)], di_blk)

        def fire(chunk, pos, b):
            si, di, hd, hs, wv, gsem, ssem = bufs[b]

            @pl.when(chunk < cend)
            def _():
                @pl.when(chunk >= c0 + 2)
                def _():
                    pltpu.make_async_copy(hd, agg_sh.at[si], ssem).wait()
                    pltpu.make_async_copy(hs, agg_sh.at[di], ssem).wait()

                for i in range(CH // 16):
                    s = pl.ds(i * 16, 16)
                    si[s] = si_blk[pl.ds(pos * CH + i * 16, 16)]
                    di[s] = di_blk[pl.ds(pos * CH + i * 16, 16)]
                pltpu.async_copy(h_h.at[di], hd, gsem)
                pltpu.async_copy(h_h.at[si], hs, gsem)
                pltpu.async_copy(w_h.at[pl.ds(chunk * CH, CH)], wv, gsem)

        def consume(chunk, b):
            si, di, hd, hs, wv, gsem, ssem = bufs[b]

            @pl.when(chunk < cend)
            def _():
                pltpu.make_async_copy(h_h.at[di], hd, gsem).wait()
                pltpu.make_async_copy(h_h.at[si], hs, gsem).wait()
                pltpu.make_async_copy(w_h.at[pl.ds(chunk * CH, CH)], wv, gsem).wait()

                def mul_body(e2, carry2):
                    e = e2 * 2
                    for ee in range(2):
                        for j in range(NF // 16):
                            s = pl.ds(j * 16, 16)
                            wj = wv[e + ee, s]
                            hd[e + ee, s] = hd[e + ee, s] * wj
                            hs[e + ee, s] = hs[e + ee, s] * wj
                    return carry2

                lax.fori_loop(0, CH // 2, mul_body, 0)
                pltpu.async_copy(hd, agg_sh.at[si], ssem, add=True)
                pltpu.async_copy(hs, agg_sh.at[di], ssem, add=True)

        refill(0)
        fire(c0, 0, 0)
        fire(c0 + 1, 1, 1)
        plsc.subcore_barrier()

        def blk_body(blk, carry):
            cb = c0 + blk * IBC

            def pair_body(p, carry2):
                j0 = 2 * p
                consume(cb + j0, 0)
                fire(cb + j0 + 2, j0 + 2, 0)
                consume(cb + j0 + 1, 1)
                fire(cb + j0 + 3, j0 + 3, 1)
                return carry2

            lax.fori_loop(0, IBC // 2 - 1, pair_body, 0)
            consume(cb + IBC - 2, 0)
            consume(cb + IBC - 1, 1)
            refill(blk + 1)
            fire(cb + IBC, 0, 0)
            fire(cb + IBC + 1, 1, 1)
            return carry

        lax.fori_loop(0, NBLK, blk_body, 0)
        nv = cend - c0
        for b in range(2):
            si, di, hd, hs, wv, gsem, ssem = bufs[b]

            @pl.when(nv > b)
            def _():
                pltpu.make_async_copy(hd, agg_sh.at[si], ssem).wait()
                pltpu.make_async_copy(hs, agg_sh.at[di], ssem).wait()

        plsc.subcore_barrier()

        for k in range(W_ROUNDS):
            wchunk = k * NS + sid

            @pl.when(wchunk < N_WCHUNKS)
            def _():
                row = wchunk * WB
                pltpu.sync_copy(agg_sh.at[pl.ds(row, WB)], hd0.at[pl.ds(0, WB)])
                pltpu.sync_copy(hd0.at[pl.ds(0, WB)], out_h.at[cid, pl.ds(row, WB)])

    f = pl.kernel(
        body,
        out_type=jax.ShapeDtypeStruct((NC, N_NODES, NF), jnp.float32),
        mesh=_sc_mesh(),
        scratch_types=[
            pltpu.VMEM((IBC * CH,), jnp.int32),
            pltpu.VMEM((IBC * CH,), jnp.int32),
            pltpu.VMEM((CH,), jnp.int32),
            pltpu.VMEM((CH,), jnp.int32),
            pltpu.VMEM((CH, NF), jnp.float32),
            pltpu.VMEM((CH, NF), jnp.float32),
            pltpu.VMEM((CH, NF), jnp.float32),
            pltpu.VMEM((CH,), jnp.int32),
            pltpu.VMEM((CH,), jnp.int32),
            pltpu.VMEM((CH, NF), jnp.float32),
            pltpu.VMEM((CH, NF), jnp.float32),
            pltpu.VMEM((CH, NF), jnp.float32),
            pltpu.VMEM_SHARED((N_NODES, NF), jnp.float32),
            pltpu.SemaphoreType.DMA,
            pltpu.SemaphoreType.DMA,
            pltpu.SemaphoreType.DMA,
            pltpu.SemaphoreType.DMA,
        ],
        name="sc_aggregate",
    )
    return f(h, w, src, dst)


_EB = 2048
_NBE = -(-N_EDGES // _EB)
E_PAD = _NBE * _EB


def _kw_body(d2_ref, fw1_ref, fb1_ref, fw2_ref, fb2_ref, w_ref):
    d = jnp.sqrt(d2_ref[...] + 1e-9).reshape(_EB, 1)
    width = CUTOFF / (NG - 1)
    offs = lax.broadcasted_iota(jnp.int32, (1, NG), 1).astype(jnp.float32) * width
    e = jnp.exp(_COEFF * (d - offs) ** 2).astype(jnp.bfloat16)
    t = _ssp(jnp.dot(e, fw1_ref[0].astype(jnp.bfloat16),
                     preferred_element_type=jnp.float32) + fb1_ref[0, 0])
    w_ref[...] = jnp.dot(t.astype(jnp.bfloat16), fw2_ref[0].astype(jnp.bfloat16),
                         preferred_element_type=jnp.float32) + fb2_ref[0, 0]


def _filters(d2, fw1, fb1, fw2, fb2):
    return pl.pallas_call(
        _kw_body,
        grid=(_NBE,),
        in_specs=[
            pl.BlockSpec((_EB,), lambda b: (b,)),
            pl.BlockSpec((1, NG, NF), lambda b: (0, 0, 0)),
            pl.BlockSpec((1, 1, NF), lambda b: (0, 0, 0)),
            pl.BlockSpec((1, NF, NF), lambda b: (0, 0, 0)),
            pl.BlockSpec((1, 1, NF), lambda b: (0, 0, 0)),
        ],
        out_specs=pl.BlockSpec((_EB, NF), lambda b: (b, 0)),
        out_shape=jax.ShapeDtypeStruct((E_PAD, NF), jnp.float32),
    )(d2, fw1, fb1, fw2, fb2)


_NBLK = 2000
_NBN = N_NODES // _NBLK


def _x0_body(r_ref, embed_ref, w0_ref, b0_ref, x_ref, h_ref):
    rr = r_ref[...]
    iot = lax.broadcasted_iota(jnp.int32, (1, 100), 1)
    onehot = (rr == iot).astype(jnp.float32)
    x = jnp.dot(onehot, embed_ref[...], preferred_element_type=jnp.float32)
    x_ref[...] = x
    h_ref[...] = jnp.dot(x, w0_ref[...], preferred_element_type=jnp.float32) + b0_ref[...]


def _embed_h0(r, embed, w0, b0):
    return pl.pallas_call(
        _x0_body,
        grid=(_NBN,),
        in_specs=[
            pl.BlockSpec((_NBLK, 1), lambda b: (b, 0)),
            pl.BlockSpec((100, NF), lambda b: (0, 0)),
            pl.BlockSpec((NF, NF), lambda b: (0, 0)),
            pl.BlockSpec((1, NF), lambda b: (0, 0)),
        ],
        out_specs=[
            pl.BlockSpec((_NBLK, NF), lambda b: (b, 0)),
            pl.BlockSpec((_NBLK, NF), lambda b: (b, 0)),
        ],
        out_shape=[
            jax.ShapeDtypeStruct((N_NODES, NF), jnp.float32),
            jax.ShapeDtypeStruct((N_NODES, NF), jnp.float32),
        ],
    )(r, embed, w0, b0)


def _dr_body(aggp_ref, x_ref, w1_ref, b1_ref, w2_ref, b2_ref, inw_ref, inb_ref,
             xo_ref, ho_ref):
    agg = aggp_ref[0] + aggp_ref[1]
    t = _ssp(jnp.dot(agg, w1_ref[...], preferred_element_type=jnp.float32) + b1_ref[...])
    dr = jnp.dot(t, w2_ref[...], preferred_element_type=jnp.float32) + b2_ref[...]
    xn = x_ref[...] + dr
    xo_ref[...] = xn
    ho_ref[...] = jnp.dot(xn, inw_ref[...], preferred_element_type=jnp.float32) + inb_ref[...]


def _update(aggp, x, w1, b1, w2, b2, inw, inb):
    return pl.pallas_call(
        _dr_body,
        grid=(_NBN,),
        in_specs=[
            pl.BlockSpec((NC, _NBLK, NF), lambda b: (0, b, 0)),
            pl.BlockSpec((_NBLK, NF), lambda b: (b, 0)),
            pl.BlockSpec((NF, NF), lambda b: (0, 0)),
            pl.BlockSpec((1, NF), lambda b: (0, 0)),
            pl.BlockSpec((NF, NF), lambda b: (0, 0)),
            pl.BlockSpec((1, NF), lambda b: (0, 0)),
            pl.BlockSpec((NF, NF), lambda b: (0, 0)),
            pl.BlockSpec((1, NF), lambda b: (0, 0)),
        ],
        out_specs=[
            pl.BlockSpec((_NBLK, NF), lambda b: (b, 0)),
            pl.BlockSpec((_NBLK, NF), lambda b: (b, 0)),
        ],
        out_shape=[
            jax.ShapeDtypeStruct((N_NODES, NF), jnp.float32),
            jax.ShapeDtypeStruct((N_NODES, NF), jnp.float32),
        ],
    )(aggp, x, w1, b1, w2, b2, inw, inb)


def _out_body(x_ref, w1_ref, b1_ref, w2_ref, b2_ref, o_ref):
    t = _ssp(jnp.dot(x_ref[...], w1_ref[...], preferred_element_type=jnp.float32) + b1_ref[...])
    y = jnp.dot(t, w2_ref[...], preferred_element_type=jnp.float32) + b2_ref[...]
    o_ref[0] = jnp.sum(y, axis=0, keepdims=True)


def _readout(x, w1, b1, w2, b2):
    return pl.pallas_call(
        _out_body,
        grid=(N_NODES // NPG,),
        in_specs=[
            pl.BlockSpec((NPG, NF), lambda g: (g, 0)),
            pl.BlockSpec((NF, NF // 2), lambda g: (0, 0)),
            pl.BlockSpec((1, NF // 2), lambda g: (0, 0)),
            pl.BlockSpec((NF // 2, 1), lambda g: (0, 0)),
            pl.BlockSpec((1, 1), lambda g: (0, 0)),
        ],
        out_specs=pl.BlockSpec((1, 1, 1), lambda g: (g, 0, 0)),
        out_shape=jax.ShapeDtypeStruct((N_NODES // NPG, 1, 1), jnp.float32),
    )(x, w1, b1, w2, b2).reshape(N_NODES // NPG, 1)


def kernel(r, xyz, a, N, embed, fw1, fb1, fw2, fb2, in2f_w, in2f_b,
           f2w1, f2b1, f2w2, f2b2, aw1_w, aw1_b, aw2_w, aw2_b):
    src = a[:, 0]
    dst = a[:, 1]
    d2 = _sc_edge_d2(xyz[:, 0], xyz[:, 1], xyz[:, 2], src, dst)
    src_p = jnp.pad(src, (0, IDXPAD - N_EDGES))
    dst_p = jnp.pad(dst, (0, IDXPAD - N_EDGES))
    fb1r = fb1.reshape(N_CONV, 1, NF)
    fb2r = fb2.reshape(N_CONV, 1, NF)

    def filt(i):
        return _filters(d2, fw1[i:i + 1], fb1r[i:i + 1], fw2[i:i + 1],
                        fb2r[i:i + 1])

    w_i = filt(0)
    x, h = _embed_h0(r, embed, in2f_w[0], in2f_b[0:1])
    for i in range(N_CONV):
        aggp = _sc_aggregate(h, w_i, src_p, dst_p)
        if i + 1 < N_CONV:
            w_i = filt(i + 1)
        nxt = (i + 1) % N_CONV
        x, h = _update(aggp, x, f2w1[i], f2b1[i:i + 1], f2w2[i], f2b2[i:i + 1],
                       in2f_w[nxt], in2f_b[nxt:nxt + 1])
    out = _readout(x, aw1_w, aw1_b.reshape(1, -1), aw2_w, aw2_b.reshape(1, 1))
    return out + NPG * (jnp.asarray(N) - NPG).astype(jnp.float32)


# speedup vs baseline: 1.0044x; 1.0044x over previous
import jax
import jax.numpy as jnp
import numpy as np
from jax import lax
from jax.experimental import pallas as pl
from jax.experimental.pallas import tpu as pltpu
from jax.experimental.pallas import tpu_sc as plsc

N_NODES = 10000
N_EDGES = 320000
NF = 128
NG = 64
N_CONV = 3
CUTOFF = 5.0
NPG = 1000
XD = 16

_OFFSETS = np.linspace(0.0, CUTOFF, NG).astype(np.float32)
_COEFF = float(-0.5 / (_OFFSETS[1] - _OFFSETS[0]) ** 2)
_LN2 = float(np.log(2.0))

NC = 2
NS = 16
NW = NC * NS
CH = 64
N_CHUNKS = N_EDGES // CH
ROUNDS = -(-N_CHUNKS // NW)
WB = 40
N_WCHUNKS = N_NODES // WB
W_ROUNDS = -(-N_WCHUNKS // NS)

import functools


@functools.cache
def _sc_mesh():
    return plsc.VectorSubcoreMesh(
        core_axis_name="c", subcore_axis_name="s", num_cores=NC, num_subcores=NS
    )


def _ssp(x):
    return jax.nn.softplus(x) - _LN2


EPT = N_EDGES // NW


def _sc_edge_d2(x0, x1, x2, src, dst):
    def body(x0_h, x1_h, x2_h, src_h, dst_h, out_h, xv0, xv1, xv2, si, di, d2v):
        cid = lax.axis_index("c")
        sid = lax.axis_index("s")
        wid = sid * NC + cid
        base = wid * EPT
        pltpu.sync_copy(x0_h, xv0)
        pltpu.sync_copy(x1_h, xv1)
        pltpu.sync_copy(x2_h, xv2)
        pltpu.sync_copy(src_h.at[pl.ds(base, EPT)], si)
        pltpu.sync_copy(dst_h.at[pl.ds(base, EPT)], di)

        def step(j, carry):
            s = pl.ds(j * 16, 16)
            ia = si[s]
            ib = di[s]
            df0 = plsc.load_gather(xv0, [ia]) - plsc.load_gather(xv0, [ib])
            df1 = plsc.load_gather(xv1, [ia]) - plsc.load_gather(xv1, [ib])
            df2 = plsc.load_gather(xv2, [ia]) - plsc.load_gather(xv2, [ib])
            d2v[s] = df0 * df0 + df1 * df1 + df2 * df2
            return carry

        lax.fori_loop(0, EPT // 16, step, 0)
        pltpu.sync_copy(d2v, out_h.at[pl.ds(base, EPT)])

    f = pl.kernel(
        body,
        out_type=jax.ShapeDtypeStruct((E_PAD,), jnp.float32),
        mesh=_sc_mesh(),
        scratch_types=[
            pltpu.VMEM((N_NODES,), jnp.float32),
            pltpu.VMEM((N_NODES,), jnp.float32),
            pltpu.VMEM((N_NODES,), jnp.float32),
            pltpu.VMEM((EPT,), jnp.int32),
            pltpu.VMEM((EPT,), jnp.int32),
            pltpu.VMEM((EPT,), jnp.float32),
        ],
        compiler_params=pltpu.CompilerParams(needs_layout_passes=False),
        name="sc_edge_d2",
    )
    return f(x0, x1, x2, src, dst)


CPT = -(-N_CHUNKS // NW)
IBC = 8
NBLK = -(-CPT // IBC)
IDXPAD = NW * (CPT + IBC) * CH


def _sc_aggregate(h, w, src, dst):
    def body(h_h, w_h, src_h, dst_h, out_h,
             si_blk, di_blk,
             si0, di0, hd0, hs0, wv0, si1, di1, hd1, hs1, wv1,
             agg_sh, gsem0, gsem1, ssem0, ssem1):
        cid = lax.axis_index("c")
        sid = lax.axis_index("s")
        wid = sid * NC + cid
        c0 = wid * CPT
        cend = jnp.minimum(c0 + CPT, N_CHUNKS)
        zero16 = jnp.zeros((16,), jnp.float32)
        bufs = ((si0, di0, hd0, hs0, wv0, gsem0, ssem0),
                (si1, di1, hd1, hs1, wv1, gsem1, ssem1))

        def zbody(e, carry):
            for j in range(NF // 16):
                hd0[e, pl.ds(j * 16, 16)] = zero16
            return carry

        lax.fori_loop(0, WB, zbody, 0)
        for k in range(W_ROUNDS):
            zchunk = k * NS + sid

            @pl.when(zchunk < N_WCHUNKS)
            def _():
                pltpu.sync_copy(
                    hd0.at[pl.ds(0, WB)],
                    agg_sh.at[pl.ds(zchunk * WB, WB)],
                )

        def refill(blk):
            off = (c0 + blk * IBC) * CH
            pltpu.sync_copy(src_h.at[pl.ds(off, IBC * CH)], si_blk)
            pltpu.sync_copy(dst_h.at[pl.ds(off, IBC * CH)], di_blk)

        def fire(chunk, pos, b):
            si, di, hd, hs, wv, gsem, ssem = bufs[b]

            @pl.when(chunk < cend)
            def _():
                @pl.when(chunk >= c0 + 2)
                def _():
                    pltpu.make_async_copy(hd, agg_sh.at[si], ssem).wait()
                    pltpu.make_async_copy(hs, agg_sh.at[di], ssem).wait()

                for i in range(CH // 16):
                    s = pl.ds(i * 16, 16)
                    si[s] = si_blk[pl.ds(pos * CH + i * 16, 16)]
                    di[s] = di_blk[pl.ds(pos * CH + i * 16, 16)]
                pltpu.async_copy(h_h.at[di], hd, gsem)
                pltpu.async_copy(h_h.at[si], hs, gsem)
                pltpu.async_copy(w_h.at[pl.ds(chunk * CH, CH)], wv, gsem)

        def consume(chunk, b):
            si, di, hd, hs, wv, gsem, ssem = bufs[b]

            @pl.when(chunk < cend)
            def _():
                pltpu.make_async_copy(h_h.at[di], hd, gsem).wait()
                pltpu.make_async_copy(h_h.at[si], hs, gsem).wait()
                pltpu.make_async_copy(w_h.at[pl.ds(chunk * CH, CH)], wv, gsem).wait()

                def mul_body(e, carry2):
                    for j in range(NF // 16):
                        s = pl.ds(j * 16, 16)
                        wj = wv[e, s]
                        hd[e, s] = hd[e, s] * wj
                        hs[e, s] = hs[e, s] * wj
                    return carry2

                lax.fori_loop(0, CH, mul_body, 0)
                pltpu.async_copy(hd, agg_sh.at[si], ssem, add=True)
                pltpu.async_copy(hs, agg_sh.at[di], ssem, add=True)

        refill(0)
        fire(c0, 0, 0)
        fire(c0 + 1, 1, 1)
        plsc.subcore_barrier()

        def blk_body(blk, carry):
            cb = c0 + blk * IBC

            def pair_body(p, carry2):
                j0 = 2 * p
                consume(cb + j0, 0)
                fire(cb + j0 + 2, j0 + 2, 0)
                consume(cb + j0 + 1, 1)
                fire(cb + j0 + 3, j0 + 3, 1)
                return carry2

            lax.fori_loop(0, IBC // 2 - 1, pair_body, 0)
            consume(cb + IBC - 2, 0)
            consume(cb + IBC - 1, 1)
            refill(blk + 1)
            fire(cb + IBC, 0, 0)
            fire(cb + IBC + 1, 1, 1)
            return carry

        lax.fori_loop(0, NBLK, blk_body, 0)
        nv = cend - c0
        for b in range(2):
            si, di, hd, hs, wv, gsem, ssem = bufs[b]

            @pl.when(nv > b)
            def _():
                pltpu.make_async_copy(hd, agg_sh.at[si], ssem).wait()
                pltpu.make_async_copy(hs, agg_sh.at[di], ssem).wait()

        plsc.subcore_barrier()

        for k in range(W_ROUNDS):
            wchunk = k * NS + sid

            @pl.when(wchunk < N_WCHUNKS)
            def _():
                row = wchunk * WB
                pltpu.sync_copy(agg_sh.at[pl.ds(row, WB)], hd0.at[pl.ds(0, WB)])
                pltpu.sync_copy(hd0.at[pl.ds(0, WB)], out_h.at[cid, pl.ds(row, WB)])

    f = pl.kernel(
        body,
        out_type=jax.ShapeDtypeStruct((NC, N_NODES, NF), jnp.float32),
        mesh=_sc_mesh(),
        scratch_types=[
            pltpu.VMEM((IBC * CH,), jnp.int32),
            pltpu.VMEM((IBC * CH,), jnp.int32),
            pltpu.VMEM((CH,), jnp.int32),
            pltpu.VMEM((CH,), jnp.int32),
            pltpu.VMEM((CH, NF), jnp.float32),
            pltpu.VMEM((CH, NF), jnp.float32),
            pltpu.VMEM((CH, NF), jnp.float32),
            pltpu.VMEM((CH,), jnp.int32),
            pltpu.VMEM((CH,), jnp.int32),
            pltpu.VMEM((CH, NF), jnp.float32),
            pltpu.VMEM((CH, NF), jnp.float32),
            pltpu.VMEM((CH, NF), jnp.float32),
            pltpu.VMEM_SHARED((N_NODES, NF), jnp.float32),
            pltpu.SemaphoreType.DMA,
            pltpu.SemaphoreType.DMA,
            pltpu.SemaphoreType.DMA,
            pltpu.SemaphoreType.DMA,
        ],
        name="sc_aggregate",
    )
    return f(h, w, src, dst)


_EB = 2048
_NBE = -(-N_EDGES // _EB)
E_PAD = _NBE * _EB


def _kw_body(d2_ref, fw1_ref, fb1_ref, fw2_ref, fb2_ref, w_ref):
    d = jnp.sqrt(d2_ref[...] + 1e-9).reshape(_EB, 1)
    width = CUTOFF / (NG - 1)
    offs = lax.broadcasted_iota(jnp.int32, (1, NG), 1).astype(jnp.float32) * width
    e = jnp.exp(_COEFF * (d - offs) ** 2)
    t = _ssp(jnp.dot(e, fw1_ref[0], preferred_element_type=jnp.float32) + fb1_ref[0, 0])
    w_ref[...] = jnp.dot(t, fw2_ref[0], preferred_element_type=jnp.float32) + fb2_ref[0, 0]


def _filters(d2, fw1, fb1, fw2, fb2):
    return pl.pallas_call(
        _kw_body,
        grid=(_NBE,),
        in_specs=[
            pl.BlockSpec((_EB,), lambda b: (b,)),
            pl.BlockSpec((1, NG, NF), lambda b: (0, 0, 0)),
            pl.BlockSpec((1, 1, NF), lambda b: (0, 0, 0)),
            pl.BlockSpec((1, NF, NF), lambda b: (0, 0, 0)),
            pl.BlockSpec((1, 1, NF), lambda b: (0, 0, 0)),
        ],
        out_specs=pl.BlockSpec((_EB, NF), lambda b: (b, 0)),
        out_shape=jax.ShapeDtypeStruct((E_PAD, NF), jnp.float32),
    )(d2, fw1, fb1, fw2, fb2)


_NBLK = 2000
_NBN = N_NODES // _NBLK


def _x0_body(r_ref, embed_ref, w0_ref, b0_ref, x_ref, h_ref):
    rr = r_ref[...]
    iot = lax.broadcasted_iota(jnp.int32, (1, 100), 1)
    onehot = (rr == iot).astype(jnp.float32)
    x = jnp.dot(onehot, embed_ref[...], preferred_element_type=jnp.float32)
    x_ref[...] = x
    h_ref[...] = jnp.dot(x, w0_ref[...], preferred_element_type=jnp.float32) + b0_ref[...]


def _embed_h0(r, embed, w0, b0):
    return pl.pallas_call(
        _x0_body,
        grid=(_NBN,),
        in_specs=[
            pl.BlockSpec((_NBLK, 1), lambda b: (b, 0)),
            pl.BlockSpec((100, NF), lambda b: (0, 0)),
            pl.BlockSpec((NF, NF), lambda b: (0, 0)),
            pl.BlockSpec((1, NF), lambda b: (0, 0)),
        ],
        out_specs=[
            pl.BlockSpec((_NBLK, NF), lambda b: (b, 0)),
            pl.BlockSpec((_NBLK, NF), lambda b: (b, 0)),
        ],
        out_shape=[
            jax.ShapeDtypeStruct((N_NODES, NF), jnp.float32),
            jax.ShapeDtypeStruct((N_NODES, NF), jnp.float32),
        ],
    )(r, embed, w0, b0)


def _dr_body(aggp_ref, x_ref, w1_ref, b1_ref, w2_ref, b2_ref, inw_ref, inb_ref,
             xo_ref, ho_ref):
    agg = aggp_ref[0] + aggp_ref[1]
    t = _ssp(jnp.dot(agg, w1_ref[...], preferred_element_type=jnp.float32) + b1_ref[...])
    dr = jnp.dot(t, w2_ref[...], preferred_element_type=jnp.float32) + b2_ref[...]
    xn = x_ref[...] + dr
    xo_ref[...] = xn
    ho_ref[...] = jnp.dot(xn, inw_ref[...], preferred_element_type=jnp.float32) + inb_ref[...]


def _update(aggp, x, w1, b1, w2, b2, inw, inb):
    return pl.pallas_call(
        _dr_body,
        grid=(_NBN,),
        in_specs=[
            pl.BlockSpec((NC, _NBLK, NF), lambda b: (0, b, 0)),
            pl.BlockSpec((_NBLK, NF), lambda b: (b, 0)),
            pl.BlockSpec((NF, NF), lambda b: (0, 0)),
            pl.BlockSpec((1, NF), lambda b: (0, 0)),
            pl.BlockSpec((NF, NF), lambda b: (0, 0)),
            pl.BlockSpec((1, NF), lambda b: (0, 0)),
            pl.BlockSpec((NF, NF), lambda b: (0, 0)),
            pl.BlockSpec((1, NF), lambda b: (0, 0)),
        ],
        out_specs=[
            pl.BlockSpec((_NBLK, NF), lambda b: (b, 0)),
            pl.BlockSpec((_NBLK, NF), lambda b: (b, 0)),
        ],
        out_shape=[
            jax.ShapeDtypeStruct((N_NODES, NF), jnp.float32),
            jax.ShapeDtypeStruct((N_NODES, NF), jnp.float32),
        ],
    )(aggp, x, w1, b1, w2, b2, inw, inb)


def _out_body(x_ref, w1_ref, b1_ref, w2_ref, b2_ref, o_ref):
    t = _ssp(jnp.dot(x_ref[...], w1_ref[...], preferred_element_type=jnp.float32) + b1_ref[...])
    y = jnp.dot(t, w2_ref[...], preferred_element_type=jnp.float32) + b2_ref[...]
    o_ref[0] = jnp.sum(y, axis=0, keepdims=True)


def _readout(x, w1, b1, w2, b2):
    return pl.pallas_call(
        _out_body,
        grid=(N_NODES // NPG,),
        in_specs=[
            pl.BlockSpec((NPG, NF), lambda g: (g, 0)),
            pl.BlockSpec((NF, NF // 2), lambda g: (0, 0)),
            pl.BlockSpec((1, NF // 2), lambda g: (0, 0)),
            pl.BlockSpec((NF // 2, 1), lambda g: (0, 0)),
            pl.BlockSpec((1, 1), lambda g: (0, 0)),
        ],
        out_specs=pl.BlockSpec((1, 1, 1), lambda g: (g, 0, 0)),
        out_shape=jax.ShapeDtypeStruct((N_NODES // NPG, 1, 1), jnp.float32),
    )(x, w1, b1, w2, b2).reshape(N_NODES // NPG, 1)


def kernel(r, xyz, a, N, embed, fw1, fb1, fw2, fb2, in2f_w, in2f_b,
           f2w1, f2b1, f2w2, f2b2, aw1_w, aw1_b, aw2_w, aw2_b):
    src = a[:, 0]
    dst = a[:, 1]
    d2 = _sc_edge_d2(xyz[:, 0], xyz[:, 1], xyz[:, 2], src, dst)
    src_p = jnp.pad(src, (0, IDXPAD - N_EDGES))
    dst_p = jnp.pad(dst, (0, IDXPAD - N_EDGES))
    fb1r = fb1.reshape(N_CONV, 1, NF)
    fb2r = fb2.reshape(N_CONV, 1, NF)

    def filt(i):
        return _filters(d2, fw1[i:i + 1], fb1r[i:i + 1], fw2[i:i + 1],
                        fb2r[i:i + 1])

    w_i = filt(0)
    x, h = _embed_h0(r, embed, in2f_w[0], in2f_b[0:1])
    for i in range(N_CONV):
        aggp = _sc_aggregate(h, w_i, src_p, dst_p)
        if i + 1 < N_CONV:
            w_i = filt(i + 1)
        nxt = (i + 1) % N_CONV
        x, h = _update(aggp, x, f2w1[i], f2b1[i:i + 1], f2w2[i], f2b2[i:i + 1],
                       in2f_w[nxt], in2f_b[nxt:nxt + 1])
    out = _readout(x, aw1_w, aw1_b.reshape(1, -1), aw2_w, aw2_b.reshape(1, 1))
    return out + NPG * (jnp.asarray(N) - NPG).astype(jnp.float32)
